# Initial kernel scaffold; baseline (speedup 1.0000x reference)
import jax
import jax.numpy as jnp
from jax.experimental import pallas as pl


def kernel(X_real, X_imag_1, X_imag_2, X_imag_3, norm_real, norm_imag_i, norm_imag_j, norm_imag_k, W1, b1, W2, b2, Wlin, blin, edge_index, query_edges):
    raise NotImplementedError("write your pallas kernel here")


# speedup vs baseline: 5.9492x; 5.9492x over previous
import functools

import jax
import jax.numpy as jnp
from jax import lax
from jax.experimental import pallas as pl
from jax.experimental.pallas import tpu as pltpu
from jax.experimental.pallas import tpu_sc as plsc

_NSC = 2
_NT = 16
_EB = 128


def _mm1_body(x_ref, w_ref, o_ref):
    o_ref[...] = jnp.concatenate(
        [jnp.dot(x_ref[c], w_ref[...], preferred_element_type=jnp.float32)
         for c in range(4)], axis=1)


def _stage1_matmul(Xs, W1, rb):
    C, N, D = Xs.shape
    H = W1.shape[1]
    return pl.pallas_call(
        _mm1_body,
        grid=(N // rb,),
        in_specs=[
            pl.BlockSpec((C, rb, D), lambda r: (0, r, 0)),
            pl.BlockSpec((D, H), lambda r: (0, 0)),
        ],
        out_specs=pl.BlockSpec((rb, C * H), lambda r: (r, 0)),
        out_shape=jax.ShapeDtypeStruct((N, C * H), jnp.float32),
    )(Xs, W1)


def _mm2_body(H, t_ref, w_ref, b_ref, o_ref):
    cols = []
    for c in range(4):
        x = t_ref[c // 2, :, (c % 2) * H:((c % 2) + 1) * H]
        x = jnp.maximum(x + b_ref[...], 0.0)
        cols.append(jnp.dot(x, w_ref[...], preferred_element_type=jnp.float32))
    o_ref[...] = jnp.concatenate(cols, axis=1)


def _stage2_matmul(T, W2, b1, rb):
    _, N, CH = T.shape
    H = W2.shape[0]
    return pl.pallas_call(
        functools.partial(_mm2_body, H),
        grid=(N // rb,),
        in_specs=[
            pl.BlockSpec((2, rb, CH), lambda r: (0, r, 0)),
            pl.BlockSpec((H, H), lambda r: (0, 0)),
            pl.BlockSpec((1, H), lambda r: (0, 0)),
        ],
        out_specs=pl.BlockSpec((rb, 4 * H), lambda r: (r, 0)),
        out_shape=jax.ShapeDtypeStruct((N, 4 * H), jnp.float32),
    )(T, W2, b1.reshape(1, H))


def _mm3_body(H, t_ref, w_ref, b_ref, o_ref):
    rb = o_ref.shape[0]
    acc = jnp.zeros((rb, 4), jnp.float32)
    for c in range(4):
        x = t_ref[c // 2, :, (c % 2) * H:((c % 2) + 1) * H]
        x = jnp.maximum(x + b_ref[...], 0.0)
        acc = acc + jnp.dot(x, w_ref[c], preferred_element_type=jnp.float32)
    o_ref[...] = jnp.concatenate(
        [acc, jnp.zeros((rb, o_ref.shape[1] - 4), jnp.float32)], axis=1)


def _stage3_logit_tables(T2, Wab, b2, rb):
    _, N, CH = T2.shape
    H = CH // 2
    return pl.pallas_call(
        functools.partial(_mm3_body, H),
        grid=(N // rb,),
        in_specs=[
            pl.BlockSpec((2, rb, CH), lambda r: (0, r, 0)),
            pl.BlockSpec((4, H, 4), lambda r: (0, 0, 0)),
            pl.BlockSpec((1, H), lambda r: (0, 0)),
        ],
        out_specs=pl.BlockSpec((rb, 128), lambda r: (r, 0)),
        out_shape=jax.ShapeDtypeStruct((N, 128), jnp.float32),
    )(T2, Wab, b2.reshape(1, H))


def _ls_body(g0_ref, g1_ref, b_ref, o_ref):
    z = g0_ref[:, 0:2] + g1_ref[:, 2:4] + b_ref[...]
    m = jnp.max(z, axis=1, keepdims=True)
    s = jnp.sum(jnp.exp(z - m), axis=1, keepdims=True)
    o_ref[...] = z - m - jnp.log(s)


def _stage4_log_softmax(G0, G1, blin, qb):
    Q, W = G0.shape
    return pl.pallas_call(
        _ls_body,
        grid=(Q // qb,),
        in_specs=[
            pl.BlockSpec((qb, W), lambda r: (r, 0)),
            pl.BlockSpec((qb, W), lambda r: (r, 0)),
            pl.BlockSpec((1, 2), lambda r: (0, 0)),
        ],
        out_specs=pl.BlockSpec((qb, 2), lambda r: (r, 0)),
        out_shape=jax.ShapeDtypeStruct((Q, 2), jnp.float32),
    )(G0, G1, blin.reshape(1, 2))


def _sc_pass(Y, src, dst, nr, ni, nj, nk, n_pad):
    N, C4 = Y.shape
    H = C4 // 4
    ch = 2 * H
    e_pad = src.shape[0]
    ept = e_pad // _NT
    nblk = ept // _EB
    rpt = n_pad // _NT
    czr = next(s for s in range(min(_EB, rpt), 0, -1) if rpt % s == 0)
    nzc = rpt // czr

    mesh = plsc.VectorSubcoreMesh(core_axis_name="c", subcore_axis_name="s")

    @functools.partial(
        pl.kernel,
        out_type=jax.ShapeDtypeStruct((_NSC, n_pad, ch), jnp.float32),
        mesh=mesh,
        scratch_types=[
            pltpu.VMEM((_EB,), jnp.int32),
            pltpu.VMEM((_EB,), jnp.int32),
            pltpu.VMEM((_EB + 16,), jnp.float32),
            pltpu.VMEM((_EB + 16,), jnp.float32),
            pltpu.VMEM((_EB + 16,), jnp.float32),
            pltpu.VMEM((_EB + 16,), jnp.float32),
            pltpu.VMEM((_EB, C4), jnp.float32),
            pltpu.VMEM((_EB, ch), jnp.float32),
            pltpu.VMEM_SHARED((n_pad, ch), jnp.float32),
            pltpu.SemaphoreType.DMA,
        ],
    )
    def k(y_hbm, src_hbm, dst_hbm, nr_hbm, ni_hbm, nj_hbm, nk_hbm, out_hbm,
          si_v, di_v, nr_v, ni_v, nj_v, nk_v, rows_v, msg_v, acc_sh, sem):
        cid = lax.axis_index("c")
        sid = lax.axis_index("s")
        is0 = cid == 0

        def zrow(r, carry):
            for kk in range(ch // 16):
                msg_v[r, pl.ds(kk * 16, 16)] = jnp.zeros((16,), jnp.float32)
            return carry
        lax.fori_loop(0, czr, zrow, 0)
        for t in range(nzc):
            pltpu.sync_copy(msg_v.at[pl.ds(0, czr)],
                            acc_sh.at[pl.ds(sid * rpt + t * czr, czr)])
        plsc.subcore_barrier()

        def blk_body(b, carry):
            base = sid * ept + b * _EB
            pltpu.sync_copy(src_hbm.at[pl.ds(base, _EB)], si_v)
            pltpu.sync_copy(dst_hbm.at[pl.ds(base, _EB)], di_v)
            pltpu.sync_copy(nr_hbm.at[pl.ds(base, _EB)],
                            nr_v.at[pl.ds(0, _EB)])
            pltpu.sync_copy(ni_hbm.at[pl.ds(base, _EB)],
                            ni_v.at[pl.ds(0, _EB)])
            pltpu.sync_copy(nj_hbm.at[pl.ds(base, _EB)],
                            nj_v.at[pl.ds(0, _EB)])
            pltpu.sync_copy(nk_hbm.at[pl.ds(base, _EB)],
                            nk_v.at[pl.ds(0, _EB)])
            pltpu.async_copy(y_hbm.at[si_v], rows_v, sem).wait()

            def ebody(e, ecarry):
                a = 0.0 - nr_v[pl.ds(e, 16)][0]
                b_ = 0.0 - ni_v[pl.ds(e, 16)][0]
                c_ = 0.0 - nj_v[pl.ds(e, 16)][0]
                d_ = 0.0 - nk_v[pl.ds(e, 16)][0]
                c00 = jnp.where(is0, a, c_)
                c01 = jnp.where(is0, -b_, d_)
                c02 = jnp.where(is0, -c_, a)
                c03 = jnp.where(is0, -d_, -b_)
                c10 = jnp.where(is0, b_, d_)
                c11 = jnp.where(is0, a, -c_)
                c12 = jnp.where(is0, -d_, b_)
                c13 = jnp.where(is0, c_, a)
                for v in range(H // 16):
                    yr = rows_v[e, pl.ds(v * 16, 16)]
                    yi = rows_v[e, pl.ds(H + v * 16, 16)]
                    yj = rows_v[e, pl.ds(2 * H + v * 16, 16)]
                    yk = rows_v[e, pl.ds(3 * H + v * 16, 16)]
                    msg_v[e, pl.ds(v * 16, 16)] = (
                        c00 * yr + c01 * yi + c02 * yj + c03 * yk)
                    msg_v[e, pl.ds(H + v * 16, 16)] = (
                        c10 * yr + c11 * yi + c12 * yj + c13 * yk)
                return ecarry
            lax.fori_loop(0, _EB, ebody, 0)
            pltpu.sync_copy(msg_v, acc_sh.at[di_v], add=True)
            return carry
        lax.fori_loop(0, nblk, blk_body, 0)

        plsc.subcore_barrier()
        nchunks = n_pad // 8
        c0 = sid * nchunks // _NT
        c1 = (sid + 1) * nchunks // _NT

        def wr(t, carry):
            pltpu.async_copy(acc_sh.at[pl.ds(t * 8, 8)],
                             out_hbm.at[cid, pl.ds(t * 8, 8)], sem)
            return carry
        lax.fori_loop(c0, c1, wr, 0)

        def wrw(t, carry):
            pltpu.make_async_copy(acc_sh.at[pl.ds(t * 8, 8)],
                                  out_hbm.at[cid, pl.ds(t * 8, 8)],
                                  sem).wait()
            return carry
        lax.fori_loop(c0, c1, wrw, 0)

    return k(Y, src, dst, nr, ni, nj, nk)


def _sc_gather(AB, q0, q1):
    Qp = q0.shape[0]
    W = AB.shape[1]
    per_w = Qp // (_NSC * _NT)
    nblk = per_w // _EB

    mesh = plsc.VectorSubcoreMesh(core_axis_name="c", subcore_axis_name="s")

    @functools.partial(
        pl.kernel,
        out_type=(jax.ShapeDtypeStruct((Qp, W), jnp.float32),
                  jax.ShapeDtypeStruct((Qp, W), jnp.float32)),
        mesh=mesh,
        scratch_types=[
            pltpu.VMEM((_EB,), jnp.int32),
            pltpu.VMEM((_EB, W), jnp.float32),
            pltpu.SemaphoreType.DMA,
        ],
    )
    def k(ab_hbm, q0_hbm, q1_hbm, g0_hbm, g1_hbm, qi_v, r_v, sem):
        cid = lax.axis_index("c")
        sid = lax.axis_index("s")
        wid = sid * _NSC + cid

        def blk(b, carry):
            base = wid * per_w + b * _EB
            pltpu.sync_copy(q0_hbm.at[pl.ds(base, _EB)], qi_v)
            pltpu.async_copy(ab_hbm.at[qi_v], r_v, sem).wait()
            pltpu.sync_copy(r_v, g0_hbm.at[pl.ds(base, _EB)])
            pltpu.sync_copy(q1_hbm.at[pl.ds(base, _EB)], qi_v)
            pltpu.async_copy(ab_hbm.at[qi_v], r_v, sem).wait()
            pltpu.sync_copy(r_v, g1_hbm.at[pl.ds(base, _EB)])
            return carry
        lax.fori_loop(0, nblk, blk, 0)

    return k(AB, q0, q1)


def kernel(X_real, X_imag_1, X_imag_2, X_imag_3, norm_real, norm_imag_i,
           norm_imag_j, norm_imag_k, W1, b1, W2, b2, Wlin, blin,
           edge_index, query_edges):
    N, D = X_real.shape
    H = W1.shape[1]
    E = norm_real.shape[0]
    Q = query_edges.shape[0]
    rb = 80

    chunk = _NT * _EB
    n_pad = ((N + _NT - 1) // _NT) * _NT
    e_pad = ((E + chunk - 1) // chunk) * chunk

    ep = e_pad - E
    zi = jnp.zeros((ep,), jnp.int32)
    zf = jnp.zeros((ep,), jnp.float32)
    src = jnp.concatenate([edge_index[0], zi])
    dst = jnp.concatenate([edge_index[1], zi])
    nr = jnp.concatenate([norm_real, zf])
    ni = jnp.concatenate([norm_imag_i, zf])
    nj = jnp.concatenate([norm_imag_j, zf])
    nk = jnp.concatenate([norm_imag_k, zf])

    Xs = jnp.stack([X_real, X_imag_1, X_imag_2, X_imag_3])
    Y1 = _stage1_matmul(Xs, W1, rb)
    T1 = _sc_pass(Y1, src, dst, nr, ni, nj, nk, n_pad)
    Y2 = _stage2_matmul(T1[:, :N, :], W2, b1, rb)
    T2 = _sc_pass(Y2, src, dst, nr, ni, nj, nk, n_pad)

    Wab = jnp.stack([
        jnp.concatenate([Wlin[2 * c * H:(2 * c + 1) * H],
                         Wlin[(2 * c + 1) * H:(2 * c + 2) * H]], axis=1)
        for c in range(4)])
    AB = _stage3_logit_tables(T2[:, :N, :], Wab, b2, rb)

    q_pad = _NSC * _NT * 8 * _EB
    zq = jnp.zeros((q_pad - Q,), jnp.int32)
    q0 = jnp.concatenate([query_edges[:, 0], zq])
    q1 = jnp.concatenate([query_edges[:, 1], zq])
    G0, G1 = _sc_gather(AB, q0, q1)

    return _stage4_log_softmax(G0, G1, blin, 256)[:Q]


# speedup vs baseline: 11.6352x; 1.9558x over previous
import functools

import jax
import jax.numpy as jnp
from jax import lax
from jax.experimental import pallas as pl
from jax.experimental.pallas import tpu as pltpu
from jax.experimental.pallas import tpu_sc as plsc

_NSC = 2
_NT = 16
_EB = 128


def _mm1_body(x_ref, w_ref, o_ref):
    o_ref[...] = jnp.concatenate(
        [jnp.dot(x_ref[c], w_ref[...], preferred_element_type=jnp.float32)
         for c in range(4)], axis=1)


def _stage1_matmul(Xs, W1, rb):
    C, N, D = Xs.shape
    H = W1.shape[1]
    return pl.pallas_call(
        _mm1_body,
        grid=(N // rb,),
        in_specs=[
            pl.BlockSpec((C, rb, D), lambda r: (0, r, 0)),
            pl.BlockSpec((D, H), lambda r: (0, 0)),
        ],
        out_specs=pl.BlockSpec((rb, C * H), lambda r: (r, 0)),
        out_shape=jax.ShapeDtypeStruct((N, C * H), jnp.float32),
    )(Xs, W1)


def _mm2_body(H, t_ref, w_ref, b_ref, o_ref):
    cols = []
    for c in range(4):
        x = t_ref[c // 2, :, (c % 2) * H:((c % 2) + 1) * H]
        x = jnp.maximum(x + b_ref[...], 0.0)
        cols.append(jnp.dot(x, w_ref[...], preferred_element_type=jnp.float32))
    o_ref[...] = jnp.concatenate(cols, axis=1)


def _stage2_matmul(T, W2, b1, rb):
    _, N, CH = T.shape
    H = W2.shape[0]
    return pl.pallas_call(
        functools.partial(_mm2_body, H),
        grid=(N // rb,),
        in_specs=[
            pl.BlockSpec((2, rb, CH), lambda r: (0, r, 0)),
            pl.BlockSpec((H, H), lambda r: (0, 0)),
            pl.BlockSpec((1, H), lambda r: (0, 0)),
        ],
        out_specs=pl.BlockSpec((rb, 4 * H), lambda r: (r, 0)),
        out_shape=jax.ShapeDtypeStruct((N, 4 * H), jnp.float32),
    )(T, W2, b1.reshape(1, H))


def _mm3_body(H, t_ref, w_ref, b_ref, o_ref):
    rb = o_ref.shape[0]
    acc = jnp.zeros((rb, 4), jnp.float32)
    for c in range(4):
        x = t_ref[c // 2, :, (c % 2) * H:((c % 2) + 1) * H]
        x = jnp.maximum(x + b_ref[...], 0.0)
        acc = acc + jnp.dot(x, w_ref[c], preferred_element_type=jnp.float32)
    o_ref[...] = jnp.concatenate(
        [acc, jnp.zeros((rb, o_ref.shape[1] - 4), jnp.float32)], axis=1)


def _stage3_logit_tables(T2, Wab, b2, rb):
    _, N, CH = T2.shape
    H = CH // 2
    return pl.pallas_call(
        functools.partial(_mm3_body, H),
        grid=(N // rb,),
        in_specs=[
            pl.BlockSpec((2, rb, CH), lambda r: (0, r, 0)),
            pl.BlockSpec((4, H, 4), lambda r: (0, 0, 0)),
            pl.BlockSpec((1, H), lambda r: (0, 0)),
        ],
        out_specs=pl.BlockSpec((rb, 128), lambda r: (r, 0)),
        out_shape=jax.ShapeDtypeStruct((N, 128), jnp.float32),
    )(T2, Wab, b2.reshape(1, H))


def _ls_body(g0_ref, g1_ref, b_ref, o_ref):
    z = g0_ref[:, 0:2] + g1_ref[:, 2:4] + b_ref[...]
    m = jnp.max(z, axis=1, keepdims=True)
    s = jnp.sum(jnp.exp(z - m), axis=1, keepdims=True)
    o_ref[...] = z - m - jnp.log(s)


def _stage4_log_softmax(G0, G1, blin, qb):
    Q, W = G0.shape
    return pl.pallas_call(
        _ls_body,
        grid=(Q // qb,),
        in_specs=[
            pl.BlockSpec((qb, W), lambda r: (r, 0)),
            pl.BlockSpec((qb, W), lambda r: (r, 0)),
            pl.BlockSpec((1, 2), lambda r: (0, 0)),
        ],
        out_specs=pl.BlockSpec((qb, 2), lambda r: (r, 0)),
        out_shape=jax.ShapeDtypeStruct((Q, 2), jnp.float32),
    )(G0, G1, blin.reshape(1, 2))


def _sc_pass(Y, eidx, enrm, n_pad):
    N, C4 = Y.shape
    H = C4 // 4
    ch = 2 * H
    TB, _, EB = eidx.shape
    nblk = TB // _NT
    rpt = n_pad // _NT
    czr = next(s for s in range(min(EB, rpt), 0, -1) if rpt % s == 0)
    nzc = rpt // czr

    mesh = plsc.VectorSubcoreMesh(core_axis_name="c", subcore_axis_name="s")

    @functools.partial(
        pl.kernel,
        out_type=jax.ShapeDtypeStruct((_NSC, n_pad, ch), jnp.float32),
        mesh=mesh,
        scratch_types=[
            pltpu.VMEM((3, 2, EB), jnp.int32),
            pltpu.VMEM((3, 4, EB), jnp.float32),
            pltpu.VMEM((2, EB, C4), jnp.float32),
            pltpu.VMEM((2, EB, ch), jnp.float32),
            pltpu.VMEM_SHARED((n_pad, ch), jnp.float32),
            pltpu.SemaphoreType.DMA,
            pltpu.SemaphoreType.DMA,
            pltpu.SemaphoreType.DMA,
        ],
    )
    def k(y_hbm, eidx_hbm, enrm_hbm, out_hbm,
          ei, en, rows, msg, acc_sh, semi, semg, sems):
        cid = lax.axis_index("c")
        sid = lax.axis_index("s")
        is0 = cid == 0
        g0 = sid * nblk

        def zrow(r, carry):
            for kk in range(ch // 16):
                msg[0, r, pl.ds(kk * 16, 16)] = jnp.zeros((16,), jnp.float32)
            return carry
        lax.fori_loop(0, czr, zrow, 0)
        for t in range(nzc):
            pltpu.sync_copy(msg.at[0, pl.ds(0, czr)],
                            acc_sh.at[pl.ds(sid * rpt + t * czr, czr)])
        plsc.subcore_barrier()

        def issue_idx(b):
            s3 = lax.rem(b, 3)
            pltpu.async_copy(eidx_hbm.at[g0 + b], ei.at[s3], semi)
            pltpu.async_copy(enrm_hbm.at[g0 + b], en.at[s3], semi)

        def wait_idx(b):
            s3 = lax.rem(b, 3)
            pltpu.make_async_copy(eidx_hbm.at[g0 + b], ei.at[s3], semi).wait()
            pltpu.make_async_copy(enrm_hbm.at[g0 + b], en.at[s3], semi).wait()

        def issue_gather(b):
            s3 = lax.rem(b, 3)
            s2 = lax.rem(b, 2)
            pltpu.async_copy(y_hbm.at[ei.at[s3, 0]], rows.at[s2], semg)

        def wait_gather(b):
            s3 = lax.rem(b, 3)
            s2 = lax.rem(b, 2)
            pltpu.make_async_copy(y_hbm.at[ei.at[s3, 0]], rows.at[s2],
                                  semg).wait()

        def issue_scatter(b):
            s3 = lax.rem(b, 3)
            s2 = lax.rem(b, 2)
            pltpu.async_copy(msg.at[s2], acc_sh.at[ei.at[s3, 1]], sems,
                             add=True)

        def wait_scatter(b):
            s3 = lax.rem(b, 3)
            s2 = lax.rem(b, 2)
            pltpu.make_async_copy(msg.at[s2], acc_sh.at[ei.at[s3, 1]],
                                  sems).wait()

        issue_idx(0)
        wait_idx(0)
        issue_gather(0)
        issue_idx(1)

        def blk_body(b, carry):
            s3 = lax.rem(b, 3)
            s2 = lax.rem(b, 2)

            @pl.when(b >= 2)
            def _():
                wait_scatter(b - 2)
            wait_gather(b)

            @pl.when(b + 1 < nblk)
            def _():
                wait_idx(b + 1)
                issue_gather(b + 1)

            @pl.when(b + 2 < nblk)
            def _():
                issue_idx(b + 2)

            def grp(g, gc):
                na = en[s3, 0, pl.ds(g * 16, 16)]
                nb = en[s3, 1, pl.ds(g * 16, 16)]
                nc = en[s3, 2, pl.ds(g * 16, 16)]
                nd = en[s3, 3, pl.ds(g * 16, 16)]
                A = 0.0 - na
                B = 0.0 - nb
                C = 0.0 - nc
                D = 0.0 - nd
                w00 = jnp.where(is0, A, C)
                w01 = jnp.where(is0, nb, D)
                w02 = jnp.where(is0, nc, A)
                w03 = jnp.where(is0, nd, nb)
                w10 = jnp.where(is0, B, D)
                w11 = jnp.where(is0, A, nc)
                w12 = jnp.where(is0, nd, B)
                w13 = jnp.where(is0, C, A)
                for l in range(16):
                    e = g * 16 + l
                    a0, a1, a2, a3 = w00[l], w01[l], w02[l], w03[l]
                    b0, b1, b2, b3 = w10[l], w11[l], w12[l], w13[l]
                    for v in range(H // 16):
                        yr = rows[s2, e, pl.ds(v * 16, 16)]
                        yi = rows[s2, e, pl.ds(H + v * 16, 16)]
                        yj = rows[s2, e, pl.ds(2 * H + v * 16, 16)]
                        yk = rows[s2, e, pl.ds(3 * H + v * 16, 16)]
                        msg[s2, e, pl.ds(v * 16, 16)] = (
                            a0 * yr + a1 * yi + a2 * yj + a3 * yk)
                        msg[s2, e, pl.ds(H + v * 16, 16)] = (
                            b0 * yr + b1 * yi + b2 * yj + b3 * yk)
                return gc
            lax.fori_loop(0, EB // 16, grp, 0)

            issue_scatter(b)
            return carry
        lax.fori_loop(0, nblk, blk_body, 0)
        wait_scatter(nblk - 2)
        wait_scatter(nblk - 1)

        plsc.subcore_barrier()
        nchunks = n_pad // 8
        c0 = sid * nchunks // _NT
        c1 = (sid + 1) * nchunks // _NT

        def wr(t, carry):
            pltpu.async_copy(acc_sh.at[pl.ds(t * 8, 8)],
                             out_hbm.at[cid, pl.ds(t * 8, 8)], semg)
            return carry
        lax.fori_loop(c0, c1, wr, 0)

        def wrw(t, carry):
            pltpu.make_async_copy(acc_sh.at[pl.ds(t * 8, 8)],
                                  out_hbm.at[cid, pl.ds(t * 8, 8)],
                                  semg).wait()
            return carry
        lax.fori_loop(c0, c1, wrw, 0)

    return k(Y, eidx, enrm)


def _sc_gather(AB, q0, q1):
    Qp = q0.shape[0]
    W = AB.shape[1]
    per_w = Qp // (_NSC * _NT)
    nblk = per_w // _EB

    mesh = plsc.VectorSubcoreMesh(core_axis_name="c", subcore_axis_name="s")

    @functools.partial(
        pl.kernel,
        out_type=(jax.ShapeDtypeStruct((Qp, W), jnp.float32),
                  jax.ShapeDtypeStruct((Qp, W), jnp.float32)),
        mesh=mesh,
        scratch_types=[
            pltpu.VMEM((_EB,), jnp.int32),
            pltpu.VMEM((_EB, W), jnp.float32),
            pltpu.SemaphoreType.DMA,
        ],
    )
    def k(ab_hbm, q0_hbm, q1_hbm, g0_hbm, g1_hbm, qi_v, r_v, sem):
        cid = lax.axis_index("c")
        sid = lax.axis_index("s")
        wid = sid * _NSC + cid

        def blk(b, carry):
            base = wid * per_w + b * _EB
            pltpu.sync_copy(q0_hbm.at[pl.ds(base, _EB)], qi_v)
            pltpu.async_copy(ab_hbm.at[qi_v], r_v, sem).wait()
            pltpu.sync_copy(r_v, g0_hbm.at[pl.ds(base, _EB)])
            pltpu.sync_copy(q1_hbm.at[pl.ds(base, _EB)], qi_v)
            pltpu.async_copy(ab_hbm.at[qi_v], r_v, sem).wait()
            pltpu.sync_copy(r_v, g1_hbm.at[pl.ds(base, _EB)])
            return carry
        lax.fori_loop(0, nblk, blk, 0)

    return k(AB, q0, q1)


def kernel(X_real, X_imag_1, X_imag_2, X_imag_3, norm_real, norm_imag_i,
           norm_imag_j, norm_imag_k, W1, b1, W2, b2, Wlin, blin,
           edge_index, query_edges):
    N, D = X_real.shape
    H = W1.shape[1]
    E = norm_real.shape[0]
    Q = query_edges.shape[0]
    rb = 80

    peb = 48
    chunk = _NT * peb
    n_pad = ((N + _NT - 1) // _NT) * _NT
    e_pad = ((E + chunk - 1) // chunk) * chunk

    ep = e_pad - E
    zi = jnp.zeros((ep,), jnp.int32)
    zf = jnp.zeros((ep,), jnp.float32)
    src = jnp.concatenate([edge_index[0], zi])
    dst = jnp.concatenate([edge_index[1], zi])
    nr = jnp.concatenate([norm_real, zf])
    ni = jnp.concatenate([norm_imag_i, zf])
    nj = jnp.concatenate([norm_imag_j, zf])
    nk = jnp.concatenate([norm_imag_k, zf])

    tb = e_pad // peb
    eidx = jnp.stack([src.reshape(tb, peb), dst.reshape(tb, peb)], axis=1)
    enrm = jnp.stack([nr.reshape(tb, peb), ni.reshape(tb, peb),
                      nj.reshape(tb, peb), nk.reshape(tb, peb)], axis=1)

    Xs = jnp.stack([X_real, X_imag_1, X_imag_2, X_imag_3])
    Y1 = _stage1_matmul(Xs, W1, rb)
    T1 = _sc_pass(Y1, eidx, enrm, n_pad)
    Y2 = _stage2_matmul(T1[:, :N, :], W2, b1, rb)
    T2 = _sc_pass(Y2, eidx, enrm, n_pad)

    Wab = jnp.stack([
        jnp.concatenate([Wlin[2 * c * H:(2 * c + 1) * H],
                         Wlin[(2 * c + 1) * H:(2 * c + 2) * H]], axis=1)
        for c in range(4)])
    AB = _stage3_logit_tables(T2[:, :N, :], Wab, b2, rb)

    q_pad = _NSC * _NT * 8 * _EB
    zq = jnp.zeros((q_pad - Q,), jnp.int32)
    q0 = jnp.concatenate([query_edges[:, 0], zq])
    q1 = jnp.concatenate([query_edges[:, 1], zq])
    G0, G1 = _sc_gather(AB, q0, q1)

    return _stage4_log_softmax(G0, G1, blin, 256)[:Q]


# speedup vs baseline: 11.6672x; 1.0028x over previous
import functools

import jax
import jax.numpy as jnp
from jax import lax
from jax.experimental import pallas as pl
from jax.experimental.pallas import tpu as pltpu
from jax.experimental.pallas import tpu_sc as plsc

_NSC = 2
_NT = 16
_EB = 128


def _mm1_body(x_ref, w_ref, o_ref):
    o_ref[...] = jnp.concatenate(
        [jnp.dot(x_ref[c], w_ref[...], preferred_element_type=jnp.float32)
         for c in range(4)], axis=1)


def _stage1_matmul(Xs, W1, rb):
    C, N, D = Xs.shape
    H = W1.shape[1]
    return pl.pallas_call(
        _mm1_body,
        grid=(N // rb,),
        in_specs=[
            pl.BlockSpec((C, rb, D), lambda r: (0, r, 0)),
            pl.BlockSpec((D, H), lambda r: (0, 0)),
        ],
        out_specs=pl.BlockSpec((rb, C * H), lambda r: (r, 0)),
        out_shape=jax.ShapeDtypeStruct((N, C * H), jnp.float32),
    )(Xs, W1)


def _mm2_body(H, t_ref, w_ref, b_ref, o_ref):
    cols = []
    for c in range(4):
        x = t_ref[c // 2, :, (c % 2) * H:((c % 2) + 1) * H]
        x = jnp.maximum(x + b_ref[...], 0.0)
        cols.append(jnp.dot(x, w_ref[...], preferred_element_type=jnp.float32))
    o_ref[...] = jnp.concatenate(cols, axis=1)


def _stage2_matmul(T, W2, b1, rb):
    _, N, CH = T.shape
    H = W2.shape[0]
    return pl.pallas_call(
        functools.partial(_mm2_body, H),
        grid=(N // rb,),
        in_specs=[
            pl.BlockSpec((2, rb, CH), lambda r: (0, r, 0)),
            pl.BlockSpec((H, H), lambda r: (0, 0)),
            pl.BlockSpec((1, H), lambda r: (0, 0)),
        ],
        out_specs=pl.BlockSpec((rb, 4 * H), lambda r: (r, 0)),
        out_shape=jax.ShapeDtypeStruct((N, 4 * H), jnp.float32),
    )(T, W2, b1.reshape(1, H))


def _mm3_body(H, t_ref, w_ref, b_ref, o_ref):
    rb = o_ref.shape[0]
    acc = jnp.zeros((rb, 4), jnp.float32)
    for c in range(4):
        x = t_ref[c // 2, :, (c % 2) * H:((c % 2) + 1) * H]
        x = jnp.maximum(x + b_ref[...], 0.0)
        acc = acc + jnp.dot(x, w_ref[c], preferred_element_type=jnp.float32)
    o_ref[...] = jnp.concatenate(
        [acc, jnp.zeros((rb, o_ref.shape[1] - 4), jnp.float32)], axis=1)


def _stage3_logit_tables(T2, Wab, b2, rb):
    _, N, CH = T2.shape
    H = CH // 2
    return pl.pallas_call(
        functools.partial(_mm3_body, H),
        grid=(N // rb,),
        in_specs=[
            pl.BlockSpec((2, rb, CH), lambda r: (0, r, 0)),
            pl.BlockSpec((4, H, 4), lambda r: (0, 0, 0)),
            pl.BlockSpec((1, H), lambda r: (0, 0)),
        ],
        out_specs=pl.BlockSpec((rb, 128), lambda r: (r, 0)),
        out_shape=jax.ShapeDtypeStruct((N, 128), jnp.float32),
    )(T2, Wab, b2.reshape(1, H))


def _ls_body(g0_ref, g1_ref, b_ref, o_ref):
    z = g0_ref[:, 0:2] + g1_ref[:, 2:4] + b_ref[...]
    m = jnp.max(z, axis=1, keepdims=True)
    s = jnp.sum(jnp.exp(z - m), axis=1, keepdims=True)
    o_ref[...] = z - m - jnp.log(s)


def _stage4_log_softmax(G0, G1, blin, qb):
    Q, W = G0.shape
    return pl.pallas_call(
        _ls_body,
        grid=(Q // qb,),
        in_specs=[
            pl.BlockSpec((qb, W), lambda r: (r, 0)),
            pl.BlockSpec((qb, W), lambda r: (r, 0)),
            pl.BlockSpec((1, 2), lambda r: (0, 0)),
        ],
        out_specs=pl.BlockSpec((qb, 2), lambda r: (r, 0)),
        out_shape=jax.ShapeDtypeStruct((Q, 2), jnp.float32),
    )(G0, G1, blin.reshape(1, 2))


def _sc_pass(Y, eidx, enrm, n_pad):
    N, C4 = Y.shape
    H = C4 // 4
    ch = 2 * H
    TB, _, EB = eidx.shape
    nblk = TB // _NT
    rpt = n_pad // _NT
    czr = next(s for s in range(min(EB, rpt), 0, -1) if rpt % s == 0)
    nzc = rpt // czr

    mesh = plsc.VectorSubcoreMesh(core_axis_name="c", subcore_axis_name="s")

    @functools.partial(
        pl.kernel,
        out_type=jax.ShapeDtypeStruct((_NSC, n_pad, ch), jnp.float32),
        mesh=mesh,
        scratch_types=[
            pltpu.VMEM((3, 2, EB), jnp.int32),
            pltpu.VMEM((3, 4, EB), jnp.float32),
            pltpu.VMEM((2, EB, C4), jnp.float32),
            pltpu.VMEM((2, EB, ch), jnp.float32),
            pltpu.VMEM_SHARED((n_pad, ch), jnp.float32),
            pltpu.SemaphoreType.DMA,
            pltpu.SemaphoreType.DMA,
            pltpu.SemaphoreType.DMA,
        ],
    )
    def k(y_hbm, eidx_hbm, enrm_hbm, out_hbm,
          ei, en, rows, msg, acc_sh, semi, semg, sems):
        cid = lax.axis_index("c")
        sid = lax.axis_index("s")
        is0 = cid == 0
        g0 = sid * nblk

        def zrow(r, carry):
            for kk in range(ch // 16):
                msg[0, r, pl.ds(kk * 16, 16)] = jnp.zeros((16,), jnp.float32)
            return carry
        lax.fori_loop(0, czr, zrow, 0)
        for t in range(nzc):
            pltpu.sync_copy(msg.at[0, pl.ds(0, czr)],
                            acc_sh.at[pl.ds(sid * rpt + t * czr, czr)])
        plsc.subcore_barrier()

        def issue_idx(b):
            s3 = lax.rem(b, 3)
            pltpu.async_copy(eidx_hbm.at[g0 + b], ei.at[s3], semi)
            pltpu.async_copy(enrm_hbm.at[g0 + b], en.at[s3], semi)

        def wait_idx(b):
            s3 = lax.rem(b, 3)
            pltpu.make_async_copy(eidx_hbm.at[g0 + b], ei.at[s3], semi).wait()
            pltpu.make_async_copy(enrm_hbm.at[g0 + b], en.at[s3], semi).wait()

        def issue_gather(b):
            s3 = lax.rem(b, 3)
            s2 = lax.rem(b, 2)
            pltpu.async_copy(y_hbm.at[ei.at[s3, 0]], rows.at[s2], semg)

        def wait_gather(b):
            s3 = lax.rem(b, 3)
            s2 = lax.rem(b, 2)
            pltpu.make_async_copy(y_hbm.at[ei.at[s3, 0]], rows.at[s2],
                                  semg).wait()

        def issue_scatter(b):
            s3 = lax.rem(b, 3)
            s2 = lax.rem(b, 2)
            pltpu.async_copy(msg.at[s2], acc_sh.at[ei.at[s3, 1]], sems,
                             add=True)

        def wait_scatter(b):
            s3 = lax.rem(b, 3)
            s2 = lax.rem(b, 2)
            pltpu.make_async_copy(msg.at[s2], acc_sh.at[ei.at[s3, 1]],
                                  sems).wait()

        issue_idx(0)
        wait_idx(0)
        issue_gather(0)
        issue_idx(1)

        def blk_body(b, carry):
            s3 = lax.rem(b, 3)
            s2 = lax.rem(b, 2)

            @pl.when(b >= 2)
            def _():
                wait_scatter(b - 2)
            wait_gather(b)

            @pl.when(b + 1 < nblk)
            def _():
                wait_idx(b + 1)
                issue_gather(b + 1)

            @pl.when(b + 2 < nblk)
            def _():
                issue_idx(b + 2)

            def grp(g, gc):
                na = en[s3, 0, pl.ds(g * 16, 16)]
                nb = en[s3, 1, pl.ds(g * 16, 16)]
                nc = en[s3, 2, pl.ds(g * 16, 16)]
                nd = en[s3, 3, pl.ds(g * 16, 16)]
                A = 0.0 - na
                B = 0.0 - nb
                C = 0.0 - nc
                D = 0.0 - nd
                w00 = jnp.where(is0, A, C)
                w01 = jnp.where(is0, nb, D)
                w02 = jnp.where(is0, nc, A)
                w03 = jnp.where(is0, nd, nb)
                w10 = jnp.where(is0, B, D)
                w11 = jnp.where(is0, A, nc)
                w12 = jnp.where(is0, nd, B)
                w13 = jnp.where(is0, C, A)
                for l in range(16):
                    e = g * 16 + l
                    a0, a1, a2, a3 = w00[l], w01[l], w02[l], w03[l]
                    b0, b1, b2, b3 = w10[l], w11[l], w12[l], w13[l]
                    for v in range(H // 16):
                        yr = rows[s2, e, pl.ds(v * 16, 16)]
                        yi = rows[s2, e, pl.ds(H + v * 16, 16)]
                        yj = rows[s2, e, pl.ds(2 * H + v * 16, 16)]
                        yk = rows[s2, e, pl.ds(3 * H + v * 16, 16)]
                        msg[s2, e, pl.ds(v * 16, 16)] = (
                            a0 * yr + a1 * yi + a2 * yj + a3 * yk)
                        msg[s2, e, pl.ds(H + v * 16, 16)] = (
                            b0 * yr + b1 * yi + b2 * yj + b3 * yk)
                return gc
            lax.fori_loop(0, EB // 16, grp, 0)

            issue_scatter(b)
            return carry
        lax.fori_loop(0, nblk, blk_body, 0)
        wait_scatter(nblk - 2)
        wait_scatter(nblk - 1)

        plsc.subcore_barrier()
        nchunks = n_pad // 8
        c0 = sid * nchunks // _NT
        c1 = (sid + 1) * nchunks // _NT

        def wr(t, carry):
            pltpu.async_copy(acc_sh.at[pl.ds(t * 8, 8)],
                             out_hbm.at[cid, pl.ds(t * 8, 8)], semg)
            return carry
        lax.fori_loop(c0, c1, wr, 0)

        def wrw(t, carry):
            pltpu.make_async_copy(acc_sh.at[pl.ds(t * 8, 8)],
                                  out_hbm.at[cid, pl.ds(t * 8, 8)],
                                  semg).wait()
            return carry
        lax.fori_loop(c0, c1, wrw, 0)

    return k(Y, eidx, enrm)


def _sc_gather(AB, q0, q1):
    Qp = q0.shape[0]
    W = AB.shape[1]
    per_w = Qp // (_NSC * _NT)
    half = per_w // 2
    nsl = half // _EB

    mesh = plsc.VectorSubcoreMesh(core_axis_name="c", subcore_axis_name="s")

    @functools.partial(
        pl.kernel,
        out_type=(jax.ShapeDtypeStruct((Qp, W), jnp.float32),
                  jax.ShapeDtypeStruct((Qp, W), jnp.float32)),
        mesh=mesh,
        scratch_types=[
            pltpu.VMEM((per_w,), jnp.int32),
            pltpu.VMEM((half, W), jnp.float32),
            pltpu.SemaphoreType.DMA,
        ],
    )
    def k(ab_hbm, q0_hbm, q1_hbm, g0_hbm, g1_hbm, qi_v, r_v, sem):
        cid = lax.axis_index("c")
        sid = lax.axis_index("s")
        wid = sid * _NSC + cid
        base = wid * per_w

        def run(q_hbm, g_hbm):
            pltpu.sync_copy(q_hbm.at[pl.ds(base, per_w)], qi_v)
            for h in range(2):
                for s in range(nsl):
                    pltpu.async_copy(
                        ab_hbm.at[qi_v.at[pl.ds(h * half + s * _EB, _EB)]],
                        r_v.at[pl.ds(s * _EB, _EB)], sem)
                for s in range(nsl):
                    pltpu.make_async_copy(
                        ab_hbm.at[qi_v.at[pl.ds(h * half + s * _EB, _EB)]],
                        r_v.at[pl.ds(s * _EB, _EB)], sem).wait()
                pltpu.sync_copy(r_v, g_hbm.at[pl.ds(base + h * half, half)])

        run(q0_hbm, g0_hbm)
        run(q1_hbm, g1_hbm)

    return k(AB, q0, q1)


def kernel(X_real, X_imag_1, X_imag_2, X_imag_3, norm_real, norm_imag_i,
           norm_imag_j, norm_imag_k, W1, b1, W2, b2, Wlin, blin,
           edge_index, query_edges):
    N, D = X_real.shape
    H = W1.shape[1]
    E = norm_real.shape[0]
    Q = query_edges.shape[0]
    rb = 80

    peb = 48
    chunk = _NT * peb
    n_pad = ((N + _NT - 1) // _NT) * _NT
    e_pad = ((E + chunk - 1) // chunk) * chunk

    ep = e_pad - E
    zi = jnp.zeros((ep,), jnp.int32)
    zf = jnp.zeros((ep,), jnp.float32)
    src = jnp.concatenate([edge_index[0], zi])
    dst = jnp.concatenate([edge_index[1], zi])
    nr = jnp.concatenate([norm_real, zf])
    ni = jnp.concatenate([norm_imag_i, zf])
    nj = jnp.concatenate([norm_imag_j, zf])
    nk = jnp.concatenate([norm_imag_k, zf])

    tb = e_pad // peb
    eidx = jnp.stack([src.reshape(tb, peb), dst.reshape(tb, peb)], axis=1)
    enrm = jnp.stack([nr.reshape(tb, peb), ni.reshape(tb, peb),
                      nj.reshape(tb, peb), nk.reshape(tb, peb)], axis=1)

    Xs = jnp.stack([X_real, X_imag_1, X_imag_2, X_imag_3])
    Y1 = _stage1_matmul(Xs, W1, rb)
    T1 = _sc_pass(Y1, eidx, enrm, n_pad)
    Y2 = _stage2_matmul(T1[:, :N, :], W2, b1, rb)
    T2 = _sc_pass(Y2, eidx, enrm, n_pad)

    Wab = jnp.stack([
        jnp.concatenate([Wlin[2 * c * H:(2 * c + 1) * H],
                         Wlin[(2 * c + 1) * H:(2 * c + 2) * H]], axis=1)
        for c in range(4)])
    AB = _stage3_logit_tables(T2[:, :N, :], Wab, b2, rb)

    q_pad = _NSC * _NT * 8 * _EB
    zq = jnp.zeros((q_pad - Q,), jnp.int32)
    q0 = jnp.concatenate([query_edges[:, 0], zq])
    q1 = jnp.concatenate([query_edges[:, 1], zq])
    G0, G1 = _sc_gather(AB, q0, q1)

    return _stage4_log_softmax(G0, G1, blin, 256)[:Q]


# speedup vs baseline: 11.6972x; 1.0026x over previous
import functools

import jax
import jax.numpy as jnp
from jax import lax
from jax.experimental import pallas as pl
from jax.experimental.pallas import tpu as pltpu
from jax.experimental.pallas import tpu_sc as plsc

_NSC = 2
_NT = 16
_EB = 128


def _mm1_body(x_ref, w_ref, o_ref):
    o_ref[...] = jnp.concatenate(
        [jnp.dot(x_ref[c], w_ref[...], preferred_element_type=jnp.float32)
         for c in range(4)], axis=1)


def _stage1_matmul(Xs, W1, rb):
    C, N, D = Xs.shape
    H = W1.shape[1]
    return pl.pallas_call(
        _mm1_body,
        grid=(N // rb,),
        in_specs=[
            pl.BlockSpec((C, rb, D), lambda r: (0, r, 0)),
            pl.BlockSpec((D, H), lambda r: (0, 0)),
        ],
        out_specs=pl.BlockSpec((rb, C * H), lambda r: (r, 0)),
        out_shape=jax.ShapeDtypeStruct((N, C * H), jnp.float32),
    )(Xs, W1)


def _mm2_body(H, t_ref, w_ref, b_ref, o_ref):
    cols = []
    for c in range(4):
        x = t_ref[c // 2, :, (c % 2) * H:((c % 2) + 1) * H]
        x = jnp.maximum(x + b_ref[...], 0.0)
        cols.append(jnp.dot(x, w_ref[...], preferred_element_type=jnp.float32))
    o_ref[...] = jnp.concatenate(cols, axis=1)


def _stage2_matmul(T, W2, b1, rb):
    _, N, CH = T.shape
    H = W2.shape[0]
    return pl.pallas_call(
        functools.partial(_mm2_body, H),
        grid=(N // rb,),
        in_specs=[
            pl.BlockSpec((2, rb, CH), lambda r: (0, r, 0)),
            pl.BlockSpec((H, H), lambda r: (0, 0)),
            pl.BlockSpec((1, H), lambda r: (0, 0)),
        ],
        out_specs=pl.BlockSpec((rb, 4 * H), lambda r: (r, 0)),
        out_shape=jax.ShapeDtypeStruct((N, 4 * H), jnp.float32),
    )(T, W2, b1.reshape(1, H))


def _mm3_body(H, t_ref, w_ref, b_ref, o_ref):
    rb = o_ref.shape[0]
    acc = jnp.zeros((rb, 4), jnp.float32)
    for c in range(4):
        x = t_ref[c // 2, :, (c % 2) * H:((c % 2) + 1) * H]
        x = jnp.maximum(x + b_ref[...], 0.0)
        acc = acc + jnp.dot(x, w_ref[c], preferred_element_type=jnp.float32)
    o_ref[...] = jnp.concatenate(
        [acc, jnp.zeros((rb, o_ref.shape[1] - 4), jnp.float32)], axis=1)


def _stage3_logit_tables(T2, Wab, b2, rb):
    _, N, CH = T2.shape
    H = CH // 2
    return pl.pallas_call(
        functools.partial(_mm3_body, H),
        grid=(N // rb,),
        in_specs=[
            pl.BlockSpec((2, rb, CH), lambda r: (0, r, 0)),
            pl.BlockSpec((4, H, 4), lambda r: (0, 0, 0)),
            pl.BlockSpec((1, H), lambda r: (0, 0)),
        ],
        out_specs=pl.BlockSpec((rb, 128), lambda r: (r, 0)),
        out_shape=jax.ShapeDtypeStruct((N, 128), jnp.float32),
    )(T2, Wab, b2.reshape(1, H))


def _ls_body(g0_ref, g1_ref, b_ref, o_ref):
    z = g0_ref[:, 0:2] + g1_ref[:, 2:4] + b_ref[...]
    m = jnp.max(z, axis=1, keepdims=True)
    s = jnp.sum(jnp.exp(z - m), axis=1, keepdims=True)
    o_ref[...] = z - m - jnp.log(s)


def _stage4_log_softmax(G0, G1, blin, qb):
    Q, W = G0.shape
    return pl.pallas_call(
        _ls_body,
        grid=(Q // qb,),
        in_specs=[
            pl.BlockSpec((qb, W), lambda r: (r, 0)),
            pl.BlockSpec((qb, W), lambda r: (r, 0)),
            pl.BlockSpec((1, 2), lambda r: (0, 0)),
        ],
        out_specs=pl.BlockSpec((qb, 2), lambda r: (r, 0)),
        out_shape=jax.ShapeDtypeStruct((Q, 2), jnp.float32),
    )(G0, G1, blin.reshape(1, 2))


def _sc_pass(Y, eidx, enrm, n_pad):
    N, C4 = Y.shape
    H = C4 // 4
    ch = 2 * H
    TB, _, EB = eidx.shape
    nblk = TB // _NT
    rpt = n_pad // _NT
    czr = next(s for s in range(min(EB, rpt), 0, -1) if rpt % s == 0)
    nzc = rpt // czr

    mesh = plsc.VectorSubcoreMesh(core_axis_name="c", subcore_axis_name="s")

    @functools.partial(
        pl.kernel,
        out_type=jax.ShapeDtypeStruct((_NSC, n_pad, ch), jnp.float32),
        mesh=mesh,
        scratch_types=[
            pltpu.VMEM((4, 2, EB), jnp.int32),
            pltpu.VMEM((4, 4, EB), jnp.float32),
            pltpu.VMEM((2, EB, C4), jnp.float32),
            pltpu.VMEM((2, EB, ch), jnp.float32),
            pltpu.VMEM_SHARED((n_pad, ch), jnp.float32),
            pltpu.SemaphoreType.DMA,
            pltpu.SemaphoreType.DMA,
            pltpu.SemaphoreType.DMA,
        ],
    )
    def k(y_hbm, eidx_hbm, enrm_hbm, out_hbm,
          ei, en, rows, msg, acc_sh, semi, semg, sems):
        cid = lax.axis_index("c")
        sid = lax.axis_index("s")
        is0 = cid == 0
        g0 = sid * nblk

        def zrow(r, carry):
            for kk in range(ch // 16):
                msg[0, r, pl.ds(kk * 16, 16)] = jnp.zeros((16,), jnp.float32)
            return carry
        lax.fori_loop(0, czr, zrow, 0)
        for t in range(nzc):
            pltpu.sync_copy(msg.at[0, pl.ds(0, czr)],
                            acc_sh.at[pl.ds(sid * rpt + t * czr, czr)])
        plsc.subcore_barrier()

        def issue_idx(b):
            s4 = lax.rem(b, 4)
            pltpu.async_copy(eidx_hbm.at[g0 + b], ei.at[s4], semi)
            pltpu.async_copy(enrm_hbm.at[g0 + b], en.at[s4], semi)

        def wait_idx(b):
            s4 = lax.rem(b, 4)
            pltpu.make_async_copy(eidx_hbm.at[g0 + b], ei.at[s4], semi).wait()
            pltpu.make_async_copy(enrm_hbm.at[g0 + b], en.at[s4], semi).wait()

        def issue_gather(b):
            s4 = lax.rem(b, 4)
            s2 = lax.rem(b, 2)
            pltpu.async_copy(y_hbm.at[ei.at[s4, 0]], rows.at[s2], semg)

        def wait_gather(b):
            s4 = lax.rem(b, 4)
            s2 = lax.rem(b, 2)
            pltpu.make_async_copy(y_hbm.at[ei.at[s4, 0]], rows.at[s2],
                                  semg).wait()

        def issue_scatter(b):
            s4 = lax.rem(b, 4)
            s2 = lax.rem(b, 2)
            pltpu.async_copy(msg.at[s2], acc_sh.at[ei.at[s4, 1]], sems,
                             add=True)

        def wait_scatter(b):
            s4 = lax.rem(b, 4)
            s2 = lax.rem(b, 2)
            pltpu.make_async_copy(msg.at[s2], acc_sh.at[ei.at[s4, 1]],
                                  sems).wait()

        issue_idx(0)
        wait_idx(0)
        issue_gather(0)
        issue_idx(1)

        def blk_body(b, carry):
            s4 = lax.rem(b, 4)
            s2 = lax.rem(b, 2)

            @pl.when(b >= 2)
            def _():
                wait_scatter(b - 2)
            wait_gather(b)

            @pl.when(b + 1 < nblk)
            def _():
                wait_idx(b + 1)
                issue_gather(b + 1)

            @pl.when(b + 2 < nblk)
            def _():
                issue_idx(b + 2)

            def grp(g, gc):
                na = en[s4, 0, pl.ds(g * 16, 16)]
                nb = en[s4, 1, pl.ds(g * 16, 16)]
                nc = en[s4, 2, pl.ds(g * 16, 16)]
                nd = en[s4, 3, pl.ds(g * 16, 16)]
                A = 0.0 - na
                B = 0.0 - nb
                C = 0.0 - nc
                D = 0.0 - nd
                w00 = jnp.where(is0, A, C)
                w01 = jnp.where(is0, nb, D)
                w02 = jnp.where(is0, nc, A)
                w03 = jnp.where(is0, nd, nb)
                w10 = jnp.where(is0, B, D)
                w11 = jnp.where(is0, A, nc)
                w12 = jnp.where(is0, nd, B)
                w13 = jnp.where(is0, C, A)
                for l in range(16):
                    e = g * 16 + l
                    a0, a1, a2, a3 = w00[l], w01[l], w02[l], w03[l]
                    b0, b1, b2, b3 = w10[l], w11[l], w12[l], w13[l]
                    for v in range(H // 16):
                        yr = rows[s2, e, pl.ds(v * 16, 16)]
                        yi = rows[s2, e, pl.ds(H + v * 16, 16)]
                        yj = rows[s2, e, pl.ds(2 * H + v * 16, 16)]
                        yk = rows[s2, e, pl.ds(3 * H + v * 16, 16)]
                        msg[s2, e, pl.ds(v * 16, 16)] = (
                            a0 * yr + a1 * yi + a2 * yj + a3 * yk)
                        msg[s2, e, pl.ds(H + v * 16, 16)] = (
                            b0 * yr + b1 * yi + b2 * yj + b3 * yk)
                return gc
            lax.fori_loop(0, EB // 16, grp, 0)

            issue_scatter(b)
            return carry
        lax.fori_loop(0, nblk, blk_body, 0)
        wait_scatter(nblk - 2)
        wait_scatter(nblk - 1)

        plsc.subcore_barrier()
        nchunks = n_pad // 8
        c0 = sid * nchunks // _NT
        c1 = (sid + 1) * nchunks // _NT

        def wr(t, carry):
            pltpu.async_copy(acc_sh.at[pl.ds(t * 8, 8)],
                             out_hbm.at[cid, pl.ds(t * 8, 8)], semg)
            return carry
        lax.fori_loop(c0, c1, wr, 0)

        def wrw(t, carry):
            pltpu.make_async_copy(acc_sh.at[pl.ds(t * 8, 8)],
                                  out_hbm.at[cid, pl.ds(t * 8, 8)],
                                  semg).wait()
            return carry
        lax.fori_loop(c0, c1, wrw, 0)

    return k(Y, eidx, enrm)


def _sc_gather(AB, q0, q1):
    Qp = q0.shape[0]
    W = AB.shape[1]
    per_w = Qp // (_NSC * _NT)
    half = per_w // 2
    nsl = half // _EB

    mesh = plsc.VectorSubcoreMesh(core_axis_name="c", subcore_axis_name="s")

    @functools.partial(
        pl.kernel,
        out_type=(jax.ShapeDtypeStruct((Qp, W), jnp.float32),
                  jax.ShapeDtypeStruct((Qp, W), jnp.float32)),
        mesh=mesh,
        scratch_types=[
            pltpu.VMEM((per_w,), jnp.int32),
            pltpu.VMEM((half, W), jnp.float32),
            pltpu.SemaphoreType.DMA,
        ],
    )
    def k(ab_hbm, q0_hbm, q1_hbm, g0_hbm, g1_hbm, qi_v, r_v, sem):
        cid = lax.axis_index("c")
        sid = lax.axis_index("s")
        wid = sid * _NSC + cid
        base = wid * per_w

        def run(q_hbm, g_hbm):
            pltpu.sync_copy(q_hbm.at[pl.ds(base, per_w)], qi_v)
            for h in range(2):
                for s in range(nsl):
                    pltpu.async_copy(
                        ab_hbm.at[qi_v.at[pl.ds(h * half + s * _EB, _EB)]],
                        r_v.at[pl.ds(s * _EB, _EB)], sem)
                for s in range(nsl):
                    pltpu.make_async_copy(
                        ab_hbm.at[qi_v.at[pl.ds(h * half + s * _EB, _EB)]],
                        r_v.at[pl.ds(s * _EB, _EB)], sem).wait()
                pltpu.sync_copy(r_v, g_hbm.at[pl.ds(base + h * half, half)])

        run(q0_hbm, g0_hbm)
        run(q1_hbm, g1_hbm)

    return k(AB, q0, q1)


def kernel(X_real, X_imag_1, X_imag_2, X_imag_3, norm_real, norm_imag_i,
           norm_imag_j, norm_imag_k, W1, b1, W2, b2, Wlin, blin,
           edge_index, query_edges):
    N, D = X_real.shape
    H = W1.shape[1]
    E = norm_real.shape[0]
    Q = query_edges.shape[0]
    rb = 80

    peb = 48
    chunk = _NT * peb
    n_pad = ((N + _NT - 1) // _NT) * _NT
    e_pad = ((E + chunk - 1) // chunk) * chunk

    ep = e_pad - E
    zi = jnp.zeros((ep,), jnp.int32)
    zf = jnp.zeros((ep,), jnp.float32)
    src = jnp.concatenate([edge_index[0], zi])
    dst = jnp.concatenate([edge_index[1], zi])
    nr = jnp.concatenate([norm_real, zf])
    ni = jnp.concatenate([norm_imag_i, zf])
    nj = jnp.concatenate([norm_imag_j, zf])
    nk = jnp.concatenate([norm_imag_k, zf])

    tb = e_pad // peb
    eidx = jnp.stack([src.reshape(tb, peb), dst.reshape(tb, peb)], axis=1)
    enrm = jnp.stack([nr.reshape(tb, peb), ni.reshape(tb, peb),
                      nj.reshape(tb, peb), nk.reshape(tb, peb)], axis=1)

    Xs = jnp.stack([X_real, X_imag_1, X_imag_2, X_imag_3])
    Y1 = _stage1_matmul(Xs, W1, rb)
    T1 = _sc_pass(Y1, eidx, enrm, n_pad)
    Y2 = _stage2_matmul(T1[:, :N, :], W2, b1, rb)
    T2 = _sc_pass(Y2, eidx, enrm, n_pad)

    Wab = jnp.stack([
        jnp.concatenate([Wlin[2 * c * H:(2 * c + 1) * H],
                         Wlin[(2 * c + 1) * H:(2 * c + 2) * H]], axis=1)
        for c in range(4)])
    AB = _stage3_logit_tables(T2[:, :N, :], Wab, b2, rb)

    q_pad = _NSC * _NT * 8 * _EB
    zq = jnp.zeros((q_pad - Q,), jnp.int32)
    q0 = jnp.concatenate([query_edges[:, 0], zq])
    q1 = jnp.concatenate([query_edges[:, 1], zq])
    G0, G1 = _sc_gather(AB, q0, q1)

    return _stage4_log_softmax(G0, G1, blin, 256)[:Q]
